# baseline (speedup 1.0000x reference)
import functools

import jax
import jax.numpy as jnp
from jax import lax
from jax.experimental import pallas as pl
from jax.experimental.pallas import tpu as pltpu
from jax.experimental.pallas import tpu_sc as plsc

NC = 2
NS = 16
L = 16
NW = NC * NS
ROWS = 128
COLS = 32768
RPW = ROWS // NW
U = 8
CHUNK = 8192
CPR = COLS // CHUNK
NCH = RPW * CPR
STEPS = CHUNK // (U * L)
I32_MAX = 2**31 - 1

_DNUMS = lax.GatherDimensionNumbers(
    offset_dims=(), collapsed_slice_dims=(0,), start_index_map=(0,)
)


def _shuf(v, perm):
    return lax.gather(
        v, perm[:, None], _DNUMS, slice_sizes=(1,),
        mode=lax.GatherScatterMode.PROMISE_IN_BOUNDS,
    )


def _chunk_max(chunk_ref, lane_iota):
    ninf = jnp.full((L,), -jnp.inf, jnp.float32)
    zero = jnp.zeros((L,), jnp.int32)

    def body(i, carry):
        maxs, steps = carry
        ib = jnp.broadcast_to(i, (L,)).astype(jnp.int32)
        new_maxs = []
        new_steps = []
        base = i * (U * L)
        for k in range(U):
            v = chunk_ref[pl.ds(base + k * L, L)]
            take = v > maxs[k]
            new_maxs.append(jnp.where(take, v, maxs[k]))
            new_steps.append(jnp.where(take, ib, steps[k]))
        return tuple(new_maxs), tuple(new_steps)

    maxs, steps = lax.fori_loop(
        0, STEPS, body, ((ninf,) * U, (zero,) * U), unroll=False
    )

    m = maxs[0]
    idx = steps[0] * (U * L) + lane_iota
    for k in range(1, U):
        idx_k = steps[k] * (U * L) + (k * L) + lane_iota
        take = (maxs[k] > m) | ((maxs[k] == m) & (idx_k < idx))
        m = jnp.where(take, maxs[k], m)
        idx = jnp.where(take, idx_k, idx)
    return m, idx


def _lane_argmax(m, idx, lane_iota):
    for sh in (1, 2, 4, 8):
        perm = lane_iota ^ sh
        m2 = _shuf(m, perm)
        idx2 = _shuf(idx, perm)
        take = (m2 > m) | ((m2 == m) & (idx2 < idx))
        m = jnp.where(take, m2, m)
        idx = jnp.where(take, idx2, idx)
    return idx


_mesh = plsc.VectorSubcoreMesh(core_axis_name="c", subcore_axis_name="s")


@functools.partial(
    pl.kernel,
    out_type=jax.ShapeDtypeStruct((NW, L), jnp.int32),
    mesh=_mesh,
    scratch_types=[
        pltpu.VMEM((2, CHUNK), jnp.float32),
        pltpu.VMEM((L,), jnp.int32),
        pltpu.SemaphoreType.DMA,
        pltpu.SemaphoreType.DMA,
    ],
)
def _argmax_sc(x_hbm, out_hbm, buf, res_v, sem0, sem1):
    wid = lax.axis_index("s") * NC + lax.axis_index("c")
    r0 = wid * RPW
    sems = (sem0, sem1)
    lane_iota = lax.iota(jnp.int32, L)

    copies = [
        pltpu.make_async_copy(
            x_hbm.at[r0 + (g // CPR), pl.ds((g % CPR) * CHUNK, CHUNK)],
            buf.at[g % 2],
            sems[g % 2],
        )
        for g in range(NCH)
    ]
    copies[0].start()
    ninf = jnp.full((L,), -jnp.inf, jnp.float32)
    zero = jnp.zeros((L,), jnp.int32)
    res = zero
    mrow, irow = ninf, zero
    for g in range(NCH):
        j, c = divmod(g, CPR)
        if g + 1 < NCH:
            copies[g + 1].start()
        copies[g].wait()
        m, idx = _chunk_max(buf.at[g % 2], lane_iota)
        idx = idx + c * CHUNK
        take = (m > mrow) | ((m == mrow) & (idx < irow))
        mrow = jnp.where(take, m, mrow)
        irow = jnp.where(take, idx, irow)
        if c == CPR - 1:
            gidx = _lane_argmax(mrow, irow, lane_iota)
            res = jnp.where(lane_iota == j, gidx, res)
            mrow, irow = ninf, zero
    res_v[...] = res
    pltpu.sync_copy(res_v, out_hbm.at[wid])


@jax.jit
def kernel(x):
    out = _argmax_sc(x)
    return out[:, :RPW].reshape(ROWS)


# speedup vs baseline: 2.5311x; 2.5311x over previous
import functools

import jax
import jax.numpy as jnp
from jax import lax
from jax.experimental import pallas as pl
from jax.experimental.pallas import tpu as pltpu

ROWS = 128
COLS = 32768
BS = 2048
SUB = BS // 128
GRID = COLS // BS
I32_MAX = 2**31 - 1


def _tc_body(x_ref, out_ref, amax_ref, astep_ref):
    j = pl.program_id(0)

    @pl.when(j == 0)
    def _init():
        amax_ref[...] = jnp.full((ROWS, 128), -jnp.inf, jnp.float32)
        astep_ref[...] = jnp.zeros((ROWS, 128), jnp.int32)

    amax = amax_ref[...]
    astep = astep_ref[...]
    for s in range(SUB):
        v = x_ref[:, s * 128:(s + 1) * 128]
        step = j * SUB + s
        take = v > amax
        amax = jnp.where(take, v, amax)
        astep = jnp.where(take, step, astep)
    amax_ref[...] = amax
    astep_ref[...] = astep

    @pl.when(j == GRID - 1)
    def _finish():
        lanes = lax.broadcasted_iota(jnp.int32, (ROWS, 128), 1)
        idx = astep * 128 + lanes
        gmax = jnp.max(amax, axis=1, keepdims=True)
        cand = jnp.where(amax == gmax, idx, I32_MAX)
        out_ref[...] = jnp.min(cand, axis=1)


_argmax_tc = pl.pallas_call(
    _tc_body,
    grid=(GRID,),
    in_specs=[pl.BlockSpec((ROWS, BS), lambda j: (0, j))],
    out_specs=pl.BlockSpec((ROWS,), lambda j: (0,)),
    out_shape=jax.ShapeDtypeStruct((ROWS,), jnp.int32),
    scratch_shapes=[
        pltpu.VMEM((ROWS, 128), jnp.float32),
        pltpu.VMEM((ROWS, 128), jnp.int32),
    ],
)


@jax.jit
def kernel(x):
    return _argmax_tc(x)


# speedup vs baseline: 3.4502x; 1.3631x over previous
import functools

import jax
import jax.numpy as jnp
from jax import lax
from jax.experimental import pallas as pl
from jax.experimental.pallas import tpu as pltpu

ROWS = 128
COLS = 32768
BS = 4096
SUB = BS // 128
GRID = COLS // BS
I32_MAX = 2**31 - 1


def _tc_body(x_ref, out_ref, amax_ref, astep_ref):
    j = pl.program_id(0)

    @pl.when(j == 0)
    def _init():
        amax_ref[...] = jnp.full((ROWS, 128), -jnp.inf, jnp.float32)
        astep_ref[...] = jnp.zeros((ROWS, 128), jnp.int32)

    amax = amax_ref[...]
    astep = astep_ref[...]
    for s in range(SUB):
        v = x_ref[:, s * 128:(s + 1) * 128]
        step = j * SUB + s
        take = v > amax
        amax = jnp.where(take, v, amax)
        astep = jnp.where(take, step, astep)
    amax_ref[...] = amax
    astep_ref[...] = astep

    @pl.when(j == GRID - 1)
    def _finish():
        lanes = lax.broadcasted_iota(jnp.int32, (ROWS, 128), 1)
        idx = astep * 128 + lanes
        gmax = jnp.max(amax, axis=1, keepdims=True)
        cand = jnp.where(amax == gmax, idx, I32_MAX)
        out_ref[...] = jnp.min(cand, axis=1)


_argmax_tc = pl.pallas_call(
    _tc_body,
    grid=(GRID,),
    in_specs=[pl.BlockSpec((ROWS, BS), lambda j: (0, j))],
    out_specs=pl.BlockSpec((ROWS,), lambda j: (0,)),
    out_shape=jax.ShapeDtypeStruct((ROWS,), jnp.int32),
    scratch_shapes=[
        pltpu.VMEM((ROWS, 128), jnp.float32),
        pltpu.VMEM((ROWS, 128), jnp.int32),
    ],
)


@jax.jit
def kernel(x):
    return _argmax_tc(x)


# speedup vs baseline: 4.0902x; 1.1855x over previous
import functools

import jax
import jax.numpy as jnp
from jax import lax
from jax.experimental import pallas as pl
from jax.experimental.pallas import tpu as pltpu

ROWS = 128
COLS = 32768
BS = 8192
SUB = BS // 128
GRID = COLS // BS
I32_MAX = 2**31 - 1


def _tc_body(x_ref, out_ref, amax_ref, astep_ref):
    j = pl.program_id(0)

    @pl.when(j == 0)
    def _init():
        amax_ref[...] = jnp.full((ROWS, 128), -jnp.inf, jnp.float32)
        astep_ref[...] = jnp.zeros((ROWS, 128), jnp.int32)

    amax = amax_ref[...]
    astep = astep_ref[...]
    for s in range(SUB):
        v = x_ref[:, s * 128:(s + 1) * 128]
        step = j * SUB + s
        take = v > amax
        amax = jnp.where(take, v, amax)
        astep = jnp.where(take, step, astep)
    amax_ref[...] = amax
    astep_ref[...] = astep

    @pl.when(j == GRID - 1)
    def _finish():
        lanes = lax.broadcasted_iota(jnp.int32, (ROWS, 128), 1)
        idx = astep * 128 + lanes
        gmax = jnp.max(amax, axis=1, keepdims=True)
        cand = jnp.where(amax == gmax, idx, I32_MAX)
        out_ref[...] = jnp.min(cand, axis=1)


_argmax_tc = pl.pallas_call(
    _tc_body,
    grid=(GRID,),
    in_specs=[pl.BlockSpec((ROWS, BS), lambda j: (0, j))],
    out_specs=pl.BlockSpec((ROWS,), lambda j: (0,)),
    out_shape=jax.ShapeDtypeStruct((ROWS,), jnp.int32),
    scratch_shapes=[
        pltpu.VMEM((ROWS, 128), jnp.float32),
        pltpu.VMEM((ROWS, 128), jnp.int32),
    ],
)


@jax.jit
def kernel(x):
    return _argmax_tc(x)
